# baseline (speedup 1.0000x reference)
import functools

import jax
import jax.numpy as jnp
from jax.experimental import pallas as pl
from jax.experimental.pallas import tpu as pltpu

N = 10000
IN_DIM = 128
OUT_DIM = 128
BM = 400


def _support_kernel(x_ref, w_ref, s_ref):
    s_ref[...] = jnp.dot(x_ref[...], w_ref[...], preferred_element_type=jnp.float32)


def _agg_kernel(a_ref, s_ref, b_ref, o_ref):
    acc = jnp.dot(a_ref[...], s_ref[...], preferred_element_type=jnp.float32)
    o_ref[...] = jnp.maximum(acc + b_ref[...], 0.0)


@functools.partial(jax.jit, static_argnames=())
def kernel(features, adj_matrix, weight, bias):
    bias2d = bias.reshape(1, OUT_DIM)
    support = pl.pallas_call(
        _support_kernel,
        out_shape=jax.ShapeDtypeStruct((N, OUT_DIM), jnp.float32),
    )(features, weight)
    out = pl.pallas_call(
        _agg_kernel,
        grid=(N // BM,),
        in_specs=[
            pl.BlockSpec((BM, N), lambda i: (i, 0)),
            pl.BlockSpec((N, OUT_DIM), lambda i: (0, 0)),
            pl.BlockSpec((1, OUT_DIM), lambda i: (0, 0)),
        ],
        out_specs=pl.BlockSpec((BM, OUT_DIM), lambda i: (i, 0)),
        out_shape=jax.ShapeDtypeStruct((N, OUT_DIM), jnp.float32),
        compiler_params=pltpu.CompilerParams(
            dimension_semantics=("parallel",),
        ),
    )(adj_matrix, support, bias2d)
    return out


# speedup vs baseline: 1.0443x; 1.0443x over previous
import functools

import jax
import jax.numpy as jnp
from jax.experimental import pallas as pl
from jax.experimental.pallas import tpu as pltpu

N = 10000
IN_DIM = 128
OUT_DIM = 128
BM = 400
BH = BM // 2


def _gcn_kernel(x_ref, w_ref, a0_ref, a1_ref, b_ref, o_ref, support_ref):
    i = pl.program_id(0)

    @pl.when(i == 0)
    def _():
        support_ref[...] = jnp.dot(
            x_ref[...], w_ref[...], preferred_element_type=jnp.float32
        )

    s = support_ref[...]
    b = b_ref[...]
    o_ref[:BH, :] = jnp.maximum(
        jnp.dot(a0_ref[...], s, preferred_element_type=jnp.float32) + b, 0.0
    )
    o_ref[BH:, :] = jnp.maximum(
        jnp.dot(a1_ref[...], s, preferred_element_type=jnp.float32) + b, 0.0
    )


@functools.partial(jax.jit, static_argnames=())
def kernel(features, adj_matrix, weight, bias):
    bias2d = bias.reshape(1, OUT_DIM)
    out = pl.pallas_call(
        _gcn_kernel,
        grid=(N // BM,),
        in_specs=[
            pl.BlockSpec((N, IN_DIM), lambda i: (0, 0)),
            pl.BlockSpec((IN_DIM, OUT_DIM), lambda i: (0, 0)),
            pl.BlockSpec((BH, N), lambda i: (2 * i, 0)),
            pl.BlockSpec((BH, N), lambda i: (2 * i + 1, 0)),
            pl.BlockSpec((1, OUT_DIM), lambda i: (0, 0)),
        ],
        out_specs=pl.BlockSpec((BM, OUT_DIM), lambda i: (i, 0)),
        out_shape=jax.ShapeDtypeStruct((N, OUT_DIM), jnp.float32),
        scratch_shapes=[pltpu.VMEM((N, OUT_DIM), jnp.float32)],
        compiler_params=pltpu.CompilerParams(
            dimension_semantics=("arbitrary",),
        ),
    )(features, weight, adj_matrix, adj_matrix, bias2d)
    return out
